# Initial kernel scaffold; baseline (speedup 1.0000x reference)
import jax
import jax.numpy as jnp
from jax.experimental import pallas as pl


def kernel(codes, codes_train, W1, b1, W2u, b2u, W2s, b2s):
    raise NotImplementedError("write your pallas kernel here")


# speedup vs baseline: 2.6372x; 2.6372x over previous
import functools

import jax
import jax.numpy as jnp
from jax import lax
from jax.experimental import pallas as pl
from jax.experimental.pallas import tpu as pltpu
from jax.experimental.pallas import tpu_sc as plsc

_K = 5
_BLK = 1024


def _topk_body(n_train, blk, codesT_ref, ct_ref, choice_ref, sel_ref,
               sv_ref, si_ref):
    t = pl.program_id(0)
    q = codesT_ref.shape[1]

    @pl.when(t == 0)
    def _init():
        sv_ref[...] = jnp.full(sv_ref.shape, jnp.inf, dtype=jnp.float32)
        si_ref[...] = jnp.zeros(si_ref.shape, dtype=jnp.int32)

    ct = ct_ref[...]
    qT = codesT_ref[...]
    dot = lax.dot_general(ct, qT, (((1,), (0,)), ((), ())),
                          preferred_element_type=jnp.float32)
    cc = jnp.sum(ct * ct, axis=1, keepdims=True)
    qq = jnp.sum(qT * qT, axis=0, keepdims=True)
    d2 = (qq - 2.0 * dot) + cc

    rows = lax.broadcasted_iota(jnp.int32, (blk, q), 0)
    limit = n_train - t * blk
    d2 = jnp.where(rows < limit, d2, jnp.inf)

    rv = [sv_ref[j:j + 1, :] for j in range(_K)]
    ri = [si_ref[j:j + 1, :] for j in range(_K)]

    x = d2
    big = jnp.int32(2 ** 30)
    for j in range(_K):
        m = jnp.min(x, axis=0, keepdims=True)
        im = jnp.min(jnp.where(x == m, rows, big), axis=0,
                     keepdims=True)
        cv, ci = m, im + t * blk
        for p in range(_K):
            keep = (rv[p] < cv) | ((rv[p] == cv) & (ri[p] < ci))
            nrv = jnp.where(keep, rv[p], cv)
            ncv = jnp.where(keep, cv, rv[p])
            nri = jnp.where(keep, ri[p], ci)
            nci = jnp.where(keep, ci, ri[p])
            rv[p], ri[p], cv, ci = nrv, nri, ncv, nci
        if j < _K - 1:
            x = jnp.where(rows == im, jnp.inf, x)

    for j in range(_K):
        sv_ref[j:j + 1, :] = rv[j]
        si_ref[j:j + 1, :] = ri[j]

    choice = choice_ref[...]
    sel = ri[0]
    for j in range(1, _K):
        sel = jnp.where(choice == j, ri[j], sel)
    sel_ref[...] = sel


def _select_topk(codesT, codes_train, choice):
    n_train, d = codes_train.shape
    q = codesT.shape[1]
    n_tiles = (n_train + _BLK - 1) // _BLK
    return pl.pallas_call(
        functools.partial(_topk_body, n_train, _BLK),
        grid=(n_tiles,),
        in_specs=[
            pl.BlockSpec((d, q), lambda t: (0, 0)),
            pl.BlockSpec((_BLK, d), lambda t: (t, 0)),
            pl.BlockSpec((1, q), lambda t: (0, 0)),
        ],
        out_specs=pl.BlockSpec((1, q), lambda t: (0, 0)),
        out_shape=jax.ShapeDtypeStruct((1, q), jnp.int32),
        scratch_shapes=[
            pltpu.VMEM((8, q), jnp.float32),
            pltpu.VMEM((8, q), jnp.int32),
        ],
    )(codesT, codes_train, choice)


def _gather_rows(table, idx):
    b, = idx.shape
    _, d = table.shape
    info = plsc.get_sparse_core_info()
    nw = info.num_cores * info.num_subcores
    b_per_w = b // nw
    mesh = plsc.VectorSubcoreMesh(core_axis_name="c", subcore_axis_name="s")

    @functools.partial(
        pl.kernel, mesh=mesh,
        compiler_params=pltpu.CompilerParams(use_tc_tiling_on_sc=False),
        out_type=jax.ShapeDtypeStruct((b, d), jnp.float32),
        scratch_types=[
            pltpu.VMEM((b_per_w,), jnp.int32),
            pltpu.VMEM((b_per_w, d), jnp.float32),
            pltpu.SemaphoreType.DMA,
        ],
    )
    def k(idx_hbm, table_hbm, out_hbm, idx_v, rows_v, sem):
        wid = lax.axis_index("s") * info.num_cores + lax.axis_index("c")
        base = wid * b_per_w
        pltpu.sync_copy(idx_hbm.at[pl.ds(base, b_per_w)], idx_v)
        pltpu.async_copy(table_hbm.at[idx_v], rows_v, sem).wait()
        pltpu.sync_copy(rows_v, out_hbm.at[pl.ds(base, b_per_w)])

    return k(idx, table)


def _mlp_body(prev_ref, w1_ref, b1_ref, w2u_ref, b2u_ref, w2s_ref, b2s_ref,
              mu_ref, ls_ref):
    prev = prev_ref[...]
    h = jnp.dot(prev, w1_ref[...], preferred_element_type=jnp.float32)
    h = jnp.maximum(h + b1_ref[...], 0.0)
    mu_ref[...] = jnp.dot(h, w2u_ref[...],
                          preferred_element_type=jnp.float32) + b2u_ref[...]
    ls_ref[...] = jnp.dot(h, w2s_ref[...],
                          preferred_element_type=jnp.float32) + b2s_ref[...]


def _mlp(prev, W1, b1, W2u, b2u, W2s, b2s):
    q, d = prev.shape
    h = W1.shape[1]
    out_sd = jax.ShapeDtypeStruct((q, d), jnp.float32)
    return pl.pallas_call(
        _mlp_body,
        out_shape=(out_sd, out_sd),
    )(prev, W1, b1.reshape(1, h), W2u, b2u.reshape(1, d),
      W2s, b2s.reshape(1, d))


def kernel(codes, codes_train, W1, b1, W2u, b2u, W2s, b2s):
    q = codes.shape[0]
    choice = jax.random.randint(jax.random.key(4543), (q,), 0, _K)
    sel = _select_topk(codes.T, codes_train, choice.astype(jnp.int32)[None, :])
    prev = _gather_rows(codes_train, sel.reshape(q))
    mu, logstd = _mlp(prev, W1, b1, W2u, b2u, W2s, b2s)
    return (mu, logstd)


# speedup vs baseline: 2.9662x; 1.1248x over previous
import functools

import jax
import jax.numpy as jnp
from jax import lax
from jax.experimental import pallas as pl
from jax.experimental.pallas import tpu as pltpu
from jax.experimental.pallas import tpu_sc as plsc

_K = 5
_BLK = 2048


def _topk_body(n_train, blk, codesT_ref, ct_ref, choice_ref, sel_ref,
               sv_ref, si_ref):
    t = pl.program_id(0)
    nt = pl.num_programs(0)
    q = codesT_ref.shape[1]

    @pl.when(t == 0)
    def _init():
        sv_ref[...] = jnp.full(sv_ref.shape, jnp.inf, dtype=jnp.float32)
        si_ref[...] = jnp.zeros(si_ref.shape, dtype=jnp.int32)

    limit = n_train - t * blk
    rows_c = lax.broadcasted_iota(jnp.int32, (blk, 1), 0)
    ct = jnp.where(rows_c < limit, ct_ref[...], 0.0)
    qT = codesT_ref[...]
    dot = lax.dot_general(ct, qT, (((1,), (0,)), ((), ())),
                          preferred_element_type=jnp.float32)
    cc = jnp.sum(ct * ct, axis=1, keepdims=True)
    cc = jnp.where(rows_c < limit, cc, jnp.inf)
    qq = jnp.sum(qT * qT, axis=0, keepdims=True) * 0.25
    d2 = (qq + dot) + cc

    rowsf = lax.broadcasted_iota(jnp.int32, (blk, q), 0).astype(jnp.float32)
    rv = [sv_ref[j:j + 1, :] for j in range(_K)]
    ri = [si_ref[j:j + 1, :] for j in range(_K)]
    x = d2
    for j in range(_K):
        m = jnp.min(x, axis=0, keepdims=True)
        w = jnp.where(x == m, rowsf, jnp.inf)
        imf = jnp.min(w, axis=0, keepdims=True)
        if j < _K - 1:
            x = jnp.where(w == imf, jnp.inf, x)
        cv, ci = m, imf.astype(jnp.int32) + t * blk
        for p in range(j, _K):
            keep = (rv[p] < cv) | ((rv[p] == cv) & (ri[p] < ci))
            nrv = jnp.where(keep, rv[p], cv)
            ncv = jnp.where(keep, cv, rv[p])
            nri = jnp.where(keep, ri[p], ci)
            nci = jnp.where(keep, ci, ri[p])
            rv[p], ri[p], cv, ci = nrv, nri, ncv, nci

    for j in range(_K):
        sv_ref[j:j + 1, :] = rv[j]
        si_ref[j:j + 1, :] = ri[j]

    @pl.when(t == nt - 1)
    def _resolve():
        choice = choice_ref[...]
        sel = ri[0]
        for j in range(1, _K):
            sel = jnp.where(choice == j, ri[j], sel)
        sel_ref[...] = sel


def _select_topk(codesT, codes_train, choice):
    n_train, d = codes_train.shape
    q = codesT.shape[1]
    n_tiles = (n_train + _BLK - 1) // _BLK
    return pl.pallas_call(
        functools.partial(_topk_body, n_train, _BLK),
        grid=(n_tiles,),
        in_specs=[
            pl.BlockSpec((d, q), lambda t: (0, 0)),
            pl.BlockSpec((_BLK, d), lambda t: (t, 0)),
            pl.BlockSpec((1, q), lambda t: (0, 0)),
        ],
        out_specs=pl.BlockSpec((1, q), lambda t: (0, 0)),
        out_shape=jax.ShapeDtypeStruct((1, q), jnp.int32),
        scratch_shapes=[
            pltpu.VMEM((8, q), jnp.float32),
            pltpu.VMEM((8, q), jnp.int32),
        ],
    )(codesT, codes_train, choice)


def _gather_rows(table, idx):
    b, = idx.shape
    _, d = table.shape
    info = plsc.get_sparse_core_info()
    nw = info.num_cores * info.num_subcores
    b_per_w = b // nw
    mesh = plsc.VectorSubcoreMesh(core_axis_name="c", subcore_axis_name="s")

    @functools.partial(
        pl.kernel, mesh=mesh,
        compiler_params=pltpu.CompilerParams(use_tc_tiling_on_sc=False),
        out_type=jax.ShapeDtypeStruct((b, d), jnp.float32),
        scratch_types=[
            pltpu.VMEM((b_per_w,), jnp.int32),
            pltpu.VMEM((b_per_w, d), jnp.float32),
            pltpu.SemaphoreType.DMA,
        ],
    )
    def k(idx_hbm, table_hbm, out_hbm, idx_v, rows_v, sem):
        wid = lax.axis_index("s") * info.num_cores + lax.axis_index("c")
        base = wid * b_per_w
        pltpu.sync_copy(idx_hbm.at[pl.ds(base, b_per_w)], idx_v)
        pltpu.async_copy(table_hbm.at[idx_v], rows_v, sem).wait()
        pltpu.sync_copy(rows_v, out_hbm.at[pl.ds(base, b_per_w)])

    return k(idx, table)


def _mlp_body(prev_ref, w1_ref, b1_ref, w2u_ref, b2u_ref, w2s_ref, b2s_ref,
              mu_ref, ls_ref):
    prev = prev_ref[...]
    h = jnp.dot(prev, w1_ref[...], preferred_element_type=jnp.float32)
    h = jnp.maximum(h + b1_ref[...], 0.0)
    mu_ref[...] = jnp.dot(h, w2u_ref[...],
                          preferred_element_type=jnp.float32) + b2u_ref[...]
    ls_ref[...] = jnp.dot(h, w2s_ref[...],
                          preferred_element_type=jnp.float32) + b2s_ref[...]


def _mlp(prev, W1, b1, W2u, b2u, W2s, b2s):
    q, d = prev.shape
    h = W1.shape[1]
    out_sd = jax.ShapeDtypeStruct((q, d), jnp.float32)
    return pl.pallas_call(
        _mlp_body,
        out_shape=(out_sd, out_sd),
    )(prev, W1, b1.reshape(1, h), W2u, b2u.reshape(1, d),
      W2s, b2s.reshape(1, d))


def kernel(codes, codes_train, W1, b1, W2u, b2u, W2s, b2s):
    q = codes.shape[0]
    choice = jax.random.randint(jax.random.key(4543), (q,), 0, _K)
    sel = _select_topk(codes.T * -2.0, codes_train,
                       choice.astype(jnp.int32)[None, :])
    prev = _gather_rows(codes_train, sel.reshape(q))
    mu, logstd = _mlp(prev, W1, b1, W2u, b2u, W2s, b2s)
    return (mu, logstd)


# speedup vs baseline: 3.1136x; 1.0497x over previous
import functools

import jax
import jax.numpy as jnp
from jax import lax
from jax.experimental import pallas as pl
from jax.experimental.pallas import tpu as pltpu
from jax.experimental.pallas import tpu_sc as plsc

_K = 5
_BLK = 1024


def _topk_body(n_train, blk, codesT_ref, ct_ref, choice_ref, sel_ref,
               sv_ref, si_ref, rf_ref, xs_ref):
    t = pl.program_id(0)
    nt = pl.num_programs(0)
    q = codesT_ref.shape[1]

    @pl.when(t == 0)
    def _init():
        sv_ref[...] = jnp.full(sv_ref.shape, jnp.inf, dtype=jnp.float32)
        si_ref[...] = jnp.zeros(si_ref.shape, dtype=jnp.int32)
        rf_ref[...] = lax.broadcasted_iota(
            jnp.int32, rf_ref.shape, 0).astype(jnp.float32)

    limit = n_train - t * blk
    rows_c = lax.broadcasted_iota(jnp.int32, (blk, 1), 0)
    ct = jnp.where(rows_c < limit, ct_ref[...], 0.0)
    qT = codesT_ref[...]
    dot = lax.dot_general(ct, qT, (((1,), (0,)), ((), ())),
                          preferred_element_type=jnp.float32)
    cc = jnp.sum(ct * ct, axis=1, keepdims=True)
    cc = jnp.where(rows_c < limit, cc, jnp.inf)
    qq = jnp.sum(qT * qT, axis=0, keepdims=True) * 0.25
    d2 = (qq + dot) + cc

    rowsf = rf_ref[...]
    tau = sv_ref[_K - 1:_K, :]

    def insert(cv, ci, j):
        for p in range(j, _K):
            rvp = sv_ref[p:p + 1, :]
            rip = si_ref[p:p + 1, :]
            keep = (rvp < cv) | ((rvp == cv) & (rip < ci))
            sv_ref[p:p + 1, :] = jnp.where(keep, rvp, cv)
            si_ref[p:p + 1, :] = jnp.where(keep, rip, ci)
            cv = jnp.where(keep, cv, rvp)
            ci = jnp.where(keep, ci, rip)

    def run_pass(j, x):
        m = jnp.min(x, axis=0, keepdims=True)
        w = jnp.where(x == m, rowsf, jnp.inf)
        imf = jnp.min(w, axis=0, keepdims=True)
        insert(m, imf.astype(jnp.int32) + t * blk, j)
        if j < _K - 1:
            @pl.when(jnp.any(m < tau))
            def _next():
                xs_ref[...] = jnp.where(w == imf, jnp.inf, x)
                run_pass(j + 1, xs_ref[...])

    run_pass(0, d2)

    @pl.when(t == nt - 1)
    def _resolve():
        choice = choice_ref[...]
        sel = si_ref[0:1, :]
        for j in range(1, _K):
            sel = jnp.where(choice == j, si_ref[j:j + 1, :], sel)
        sel_ref[...] = sel


def _select_topk(codesT, codes_train, choice):
    n_train, d = codes_train.shape
    q = codesT.shape[1]
    n_tiles = (n_train + _BLK - 1) // _BLK
    return pl.pallas_call(
        functools.partial(_topk_body, n_train, _BLK),
        grid=(n_tiles,),
        in_specs=[
            pl.BlockSpec((d, q), lambda t: (0, 0)),
            pl.BlockSpec((_BLK, d), lambda t: (t, 0)),
            pl.BlockSpec((1, q), lambda t: (0, 0)),
        ],
        out_specs=pl.BlockSpec((1, q), lambda t: (0, 0)),
        out_shape=jax.ShapeDtypeStruct((1, q), jnp.int32),
        scratch_shapes=[
            pltpu.VMEM((8, q), jnp.float32),
            pltpu.VMEM((8, q), jnp.int32),
            pltpu.VMEM((_BLK, q), jnp.float32),
            pltpu.VMEM((_BLK, q), jnp.float32),
        ],
    )(codesT, codes_train, choice)


def _gather_rows(table, idx):
    b, = idx.shape
    _, d = table.shape
    info = plsc.get_sparse_core_info()
    nw = info.num_cores * info.num_subcores
    b_per_w = b // nw
    mesh = plsc.VectorSubcoreMesh(core_axis_name="c", subcore_axis_name="s")

    @functools.partial(
        pl.kernel, mesh=mesh,
        compiler_params=pltpu.CompilerParams(use_tc_tiling_on_sc=False),
        out_type=jax.ShapeDtypeStruct((b, d), jnp.float32),
        scratch_types=[
            pltpu.VMEM((b_per_w,), jnp.int32),
            pltpu.VMEM((b_per_w, d), jnp.float32),
            pltpu.SemaphoreType.DMA,
        ],
    )
    def k(idx_hbm, table_hbm, out_hbm, idx_v, rows_v, sem):
        wid = lax.axis_index("s") * info.num_cores + lax.axis_index("c")
        base = wid * b_per_w
        pltpu.sync_copy(idx_hbm.at[pl.ds(base, b_per_w)], idx_v)
        pltpu.async_copy(table_hbm.at[idx_v], rows_v, sem).wait()
        pltpu.sync_copy(rows_v, out_hbm.at[pl.ds(base, b_per_w)])

    return k(idx, table)


def _mlp_body(prev_ref, w1_ref, b1_ref, w2u_ref, b2u_ref, w2s_ref, b2s_ref,
              mu_ref, ls_ref):
    prev = prev_ref[...]
    h = jnp.dot(prev, w1_ref[...], preferred_element_type=jnp.float32)
    h = jnp.maximum(h + b1_ref[...], 0.0)
    mu_ref[...] = jnp.dot(h, w2u_ref[...],
                          preferred_element_type=jnp.float32) + b2u_ref[...]
    ls_ref[...] = jnp.dot(h, w2s_ref[...],
                          preferred_element_type=jnp.float32) + b2s_ref[...]


def _mlp(prev, W1, b1, W2u, b2u, W2s, b2s):
    q, d = prev.shape
    h = W1.shape[1]
    out_sd = jax.ShapeDtypeStruct((q, d), jnp.float32)
    return pl.pallas_call(
        _mlp_body,
        out_shape=(out_sd, out_sd),
    )(prev, W1, b1.reshape(1, h), W2u, b2u.reshape(1, d),
      W2s, b2s.reshape(1, d))


def kernel(codes, codes_train, W1, b1, W2u, b2u, W2s, b2s):
    q = codes.shape[0]
    choice = jax.random.randint(jax.random.key(4543), (q,), 0, _K)
    sel = _select_topk(codes.T * -2.0, codes_train,
                       choice.astype(jnp.int32)[None, :])
    prev = _gather_rows(codes_train, sel.reshape(q))
    mu, logstd = _mlp(prev, W1, b1, W2u, b2u, W2s, b2s)
    return (mu, logstd)
